# Initial kernel scaffold; baseline (speedup 1.0000x reference)
import jax
import jax.numpy as jnp
from jax.experimental import pallas as pl


def kernel(hidden_states, Wg, bg, Wn, bn, W_in, b_in, W_out, b_out):
    raise NotImplementedError("write your pallas kernel here")


# speedup vs baseline: 1.5968x; 1.5968x over previous
import functools

import jax
import jax.numpy as jnp
from jax.experimental import pallas as pl
from jax.experimental.pallas import tpu as pltpu

D = 1024
FF = 4096
E = 8
K = 2
T = 2048
TT = 256
FJ = 512
_INV_SQRT2 = 0.7071067811865476


def _router_body(tokens_ref, wg_ref, bg_ref, wn_ref, bn_ref, noise_ref,
                 gate_ref, noisy_ref, comb_ref):
    x = tokens_ref[...]
    gate = jnp.dot(x, wg_ref[...], preferred_element_type=jnp.float32) + bg_ref[...]
    nl = jnp.dot(x, wn_ref[...], preferred_element_type=jnp.float32) + bn_ref[...]
    sp = jnp.maximum(nl, 0.0) + jnp.log1p(jnp.exp(-jnp.abs(nl)))
    noisy = gate + noise_ref[...] * sp
    gate_ref[...] = gate
    noisy_ref[...] = noisy
    m = jnp.max(noisy, axis=1, keepdims=True)
    ex = jnp.exp(noisy - m)
    rw = ex / jnp.sum(ex, axis=1, keepdims=True)
    lane = jax.lax.broadcasted_iota(jnp.int32, rw.shape, 1)
    m1 = jnp.max(rw, axis=1, keepdims=True)
    idx1 = jnp.min(jnp.where(rw == m1, lane, E), axis=1, keepdims=True)
    sel1 = lane == idx1
    rw_rest = jnp.where(sel1, -jnp.inf, rw)
    m2 = jnp.max(rw_rest, axis=1, keepdims=True)
    idx2 = jnp.min(jnp.where(rw_rest == m2, lane, E), axis=1, keepdims=True)
    sel2 = lane == idx2
    denom = m1 + m2
    comb_ref[...] = jnp.where(sel1, m1 / denom, 0.0) + jnp.where(sel2, m2 / denom, 0.0)


def _dense_body(tokens_ref, comb_ref, win_ref, bin_ref, wout_ref, bout_ref,
                out_ref, acc_ref):
    e = pl.program_id(1)

    @pl.when(e == 0)
    def _():
        acc_ref[...] = jnp.zeros_like(acc_ref)

    x = tokens_ref[...].astype(jnp.bfloat16)
    lane = jax.lax.broadcasted_iota(jnp.int32, comb_ref.shape, 1)
    comb_col = jnp.sum(jnp.where(lane == e, comb_ref[...], 0.0), axis=1,
                       keepdims=True)

    def body(j, _):
        win = win_ref[0, :, pl.ds(j * FJ, FJ)]
        h = jnp.dot(x, win, preferred_element_type=jnp.float32)
        h = h + bin_ref[0, 0, pl.ds(j * FJ, FJ)][None, :]
        h = h * 0.5 * (1.0 + jax.lax.erf(h * _INV_SQRT2))
        wout = wout_ref[0, pl.ds(j * FJ, FJ), :]
        acc_ref[...] += comb_col * jnp.dot(h.astype(jnp.bfloat16), wout,
                                           preferred_element_type=jnp.float32)
        return 0

    jax.lax.fori_loop(0, FF // FJ, body, 0)
    acc_ref[...] += comb_col * bout_ref[0, 0, :][None, :]

    @pl.when(e == E - 1)
    def _():
        out_ref[...] = acc_ref[...]


@jax.jit
def kernel(hidden_states, Wg, bg, Wn, bn, W_in, b_in, W_out, b_out):
    B, S, _ = hidden_states.shape
    tokens = hidden_states.reshape(-1, D)
    noise = jax.random.normal(jax.random.key(42), (T, E), dtype=jnp.float32)

    gate, noisy, comb = pl.pallas_call(
        _router_body,
        out_shape=(
            jax.ShapeDtypeStruct((T, E), jnp.float32),
            jax.ShapeDtypeStruct((T, E), jnp.float32),
            jax.ShapeDtypeStruct((T, E), jnp.float32),
        ),
    )(tokens, Wg, bg.reshape(1, E), Wn, bn.reshape(1, E), noise)

    win_bf = W_in.astype(jnp.bfloat16)
    wout_bf = W_out.astype(jnp.bfloat16)

    out = pl.pallas_call(
        _dense_body,
        grid=(T // TT, E),
        in_specs=[
            pl.BlockSpec((TT, D), lambda t, e: (t, 0)),
            pl.BlockSpec((TT, E), lambda t, e: (t, 0)),
            pl.BlockSpec((1, D, FF), lambda t, e: (e, 0, 0)),
            pl.BlockSpec((1, 1, FF), lambda t, e: (e, 0, 0)),
            pl.BlockSpec((1, FF, D), lambda t, e: (e, 0, 0)),
            pl.BlockSpec((1, 1, D), lambda t, e: (e, 0, 0)),
        ],
        out_specs=pl.BlockSpec((TT, D), lambda t, e: (t, 0)),
        out_shape=jax.ShapeDtypeStruct((T, D), jnp.float32),
        scratch_shapes=[pltpu.VMEM((TT, D), jnp.float32)],
    )(tokens, comb, win_bf, b_in.reshape(E, 1, FF), wout_bf,
      b_out.reshape(E, 1, D))

    return (out.reshape(B, S, D), noisy, gate)


# speedup vs baseline: 2.0931x; 1.3108x over previous
import functools

import jax
import jax.numpy as jnp
from jax import lax
from jax.experimental import pallas as pl
from jax.experimental.pallas import tpu as pltpu
from jax.experimental.pallas import tpu_sc as plsc

D = 1024
FF = 4096
E = 8
K = 2
T = 2048
TM = 256
NB_MAX = (T * K) // TM + E
NSLOTS = NB_MAX * TM
FJ = 512
NP = T * K
_INV_SQRT2 = 0.7071067811865476

NC, NS, LANES = 2, 16, 16
NW = NC * NS
_GCH = 64
_ROWS_PER_W = NSLOTS // NW
_CCH = 32
_TOK_PER_W = T // NW


def _router_body(tokens_ref, wg_ref, bg_ref, wn_ref, bn_ref, noise_ref,
                 gate_ref, noisy_ref, slot_ref, w_ref, eb_ref, nbtot_ref):
    x = tokens_ref[...]
    gate = jnp.dot(x, wg_ref[...], preferred_element_type=jnp.float32) + bg_ref[...]
    nl = jnp.dot(x, wn_ref[...], preferred_element_type=jnp.float32) + bn_ref[...]
    sp = jnp.maximum(nl, 0.0) + jnp.log1p(jnp.exp(-jnp.abs(nl)))
    noisy = gate + noise_ref[...] * sp
    gate_ref[...] = gate
    noisy_ref[...] = noisy
    m = jnp.max(noisy, axis=1, keepdims=True)
    ex = jnp.exp(noisy - m)
    rw = ex / jnp.sum(ex, axis=1, keepdims=True)
    lane = lax.broadcasted_iota(jnp.int32, rw.shape, 1)
    m1 = jnp.max(rw, axis=1, keepdims=True)
    idx1 = jnp.min(jnp.where(rw == m1, lane, E), axis=1, keepdims=True)
    sel1 = lane == idx1
    rw_rest = jnp.where(sel1, -jnp.inf, rw)
    m2 = jnp.max(rw_rest, axis=1, keepdims=True)
    idx2 = jnp.min(jnp.where(rw_rest == m2, lane, E), axis=1, keepdims=True)
    sel2 = lane == idx2
    denom = m1 + m2
    w_ref[...] = jnp.concatenate([m1 / denom, m2 / denom], axis=1)

    sel = (sel1 | sel2).astype(jnp.bfloat16)
    r_i = lax.broadcasted_iota(jnp.int32, (T, T), 0)
    c_i = lax.broadcasted_iota(jnp.int32, (T, T), 1)
    ltri = (r_i >= c_i).astype(jnp.bfloat16)
    ranks = jnp.dot(ltri, sel, preferred_element_type=jnp.float32)
    counts = ranks[T - 1:T, :]
    nb = jnp.floor((counts + (TM - 1)) * (1.0 / TM))
    a_i = lax.broadcasted_iota(jnp.int32, (E, E), 0)
    b_i = lax.broadcasted_iota(jnp.int32, (E, E), 1)
    utri = (a_i <= b_i).astype(jnp.float32)
    nb_cum = jnp.dot(nb, utri, preferred_element_type=jnp.float32)
    base = (nb_cum - nb) * TM
    nbtot = nb_cum[0:1, E - 1:E]
    nbtot_ref[...] = nbtot.astype(jnp.int32)

    def pick(sel_k):
        rank_k = jnp.sum(jnp.where(sel_k, ranks, 0.0), axis=1, keepdims=True)
        base_k = jnp.sum(jnp.where(sel_k, base, 0.0), axis=1, keepdims=True)
        return (base_k + rank_k - 1.0).astype(jnp.int32)

    slot_ref[...] = jnp.concatenate([pick(sel1), pick(sel2)], axis=1)

    ends = nb_cum * TM
    bb = lax.broadcasted_iota(jnp.int32, (NB_MAX, 1), 0).astype(jnp.float32)
    bmin = jnp.minimum(bb, nbtot - 1.0) * TM
    eb_ref[...] = jnp.sum((ends <= bmin).astype(jnp.int32), axis=1,
                          keepdims=True)


def _run_router(tokens, Wg, bg, Wn, bn, noise):
    return pl.pallas_call(
        _router_body,
        out_shape=(
            jax.ShapeDtypeStruct((T, E), jnp.float32),
            jax.ShapeDtypeStruct((T, E), jnp.float32),
            jax.ShapeDtypeStruct((T, K), jnp.int32),
            jax.ShapeDtypeStruct((T, K), jnp.float32),
            jax.ShapeDtypeStruct((NB_MAX, 1), jnp.int32),
            jax.ShapeDtypeStruct((1, 1), jnp.int32),
        ),
    )(tokens, Wg, bg.reshape(1, E), Wn, bn.reshape(1, E), noise)


def _gmm_body(eb_ref, nbtot_ref, x_ref, wps_ref, win_ref, bin_ref, wout_ref,
              bout_ref, h_ref, acc_ref):
    b = pl.program_id(0)

    @pl.when(b < nbtot_ref[0])
    def _():
        x = x_ref[...].astype(jnp.bfloat16)

        def body(j, carry):
            sl = pl.ds(j * FJ, FJ)
            h1 = jnp.dot(x, win_ref[0, :, sl],
                         preferred_element_type=jnp.float32)
            h1 = h1 + bin_ref[0, 0, sl][None, :]
            h1 = h1 * 0.5 * (1.0 + lax.erf(h1 * _INV_SQRT2))
            acc = jnp.dot(h1.astype(jnp.bfloat16), wout_ref[0, sl, :],
                          preferred_element_type=jnp.float32)

            @pl.when(j == 0)
            def _():
                acc_ref[...] = acc

            @pl.when(j > 0)
            def _():
                acc_ref[...] += acc

            return carry

        lax.fori_loop(0, FF // FJ, body, 0)
        h_ref[...] = wps_ref[...] * (acc_ref[...] + bout_ref[0, 0, :][None, :])


def _run_gmm(eb, nbtot, x_sorted, w_slot, win_bf, b_in3, wout_bf, b_out3):
    grid_spec = pltpu.PrefetchScalarGridSpec(
        num_scalar_prefetch=2,
        grid=(NB_MAX,),
        in_specs=[
            pl.BlockSpec((TM, D), lambda b, eb, nt: (b, 0)),
            pl.BlockSpec((TM, 1), lambda b, eb, nt: (b, 0)),
            pl.BlockSpec((1, D, FF), lambda b, eb, nt: (eb[b], 0, 0)),
            pl.BlockSpec((1, 1, FF), lambda b, eb, nt: (eb[b], 0, 0)),
            pl.BlockSpec((1, FF, D), lambda b, eb, nt: (eb[b], 0, 0)),
            pl.BlockSpec((1, 1, D), lambda b, eb, nt: (eb[b], 0, 0)),
        ],
        out_specs=pl.BlockSpec((TM, D), lambda b, eb, nt: (b, 0)),
        scratch_shapes=[pltpu.VMEM((TM, D), jnp.float32)],
    )
    return pl.pallas_call(
        _gmm_body,
        grid_spec=grid_spec,
        out_shape=jax.ShapeDtypeStruct((NSLOTS, D), jnp.float32),
    )(eb, nbtot, x_sorted, w_slot, win_bf, b_in3, wout_bf, b_out3)


@functools.cache
def _sc_kernels():
    mesh = plsc.VectorSubcoreMesh(core_axis_name="c", subcore_axis_name="s",
                                  num_cores=NC, num_subcores=NS)

    @functools.partial(
        pl.kernel,
        out_type=(jax.ShapeDtypeStruct((NSLOTS,), jnp.int32),
                  jax.ShapeDtypeStruct((NSLOTS,), jnp.float32)),
        mesh=mesh,
        compiler_params=pltpu.CompilerParams(needs_layout_passes=False),
        scratch_types=[pltpu.VMEM((NP,), jnp.int32),
                       pltpu.VMEM((NP,), jnp.float32),
                       pltpu.VMEM((NSLOTS,), jnp.int32),
                       pltpu.VMEM((NSLOTS,), jnp.float32)],
    )
    def sc_invert(slot_hbm, w_hbm, tok_hbm, wslot_hbm, slot_v, w_v, tok_v,
                  wslot_v):
        cid = lax.axis_index("c")
        sid = lax.axis_index("s")

        @pl.when(jnp.logical_and(cid == 0, sid == 0))
        def _():
            pltpu.sync_copy(slot_hbm, slot_v)
            pltpu.sync_copy(w_hbm, w_v)
            zi = jnp.zeros((LANES,), jnp.int32)
            zf = jnp.zeros((LANES,), jnp.float32)

            def init(i, carry):
                tok_v[pl.ds(i * LANES, LANES)] = zi
                wslot_v[pl.ds(i * LANES, LANES)] = zf
                return carry

            lax.fori_loop(0, NSLOTS // LANES, init, 0)
            lanes = lax.iota(jnp.int32, LANES)

            def body(i, carry):
                sl = pl.ds(i * LANES, LANES)
                idx = slot_v[sl]
                plsc.store_scatter(tok_v, [idx], (i * LANES + lanes) >> 1)
                plsc.store_scatter(wslot_v, [idx], w_v[sl])
                return carry

            lax.fori_loop(0, NP // LANES, body, 0)
            pltpu.sync_copy(tok_v, tok_hbm)
            pltpu.sync_copy(wslot_v, wslot_hbm)

    @functools.partial(
        pl.kernel,
        out_type=jax.ShapeDtypeStruct((NSLOTS, D), jnp.float32),
        mesh=mesh,
        compiler_params=pltpu.CompilerParams(needs_layout_passes=False),
        scratch_types=[pltpu.VMEM((_GCH,), jnp.int32),
                       pltpu.VMEM((_GCH, D), jnp.float32),
                       pltpu.SemaphoreType.DMA],
    )
    def sc_gather_tokens(tok_slot_hbm, tokens_hbm, x_hbm, idx_v, rows_v, sem):
        wid = lax.axis_index("s") * NC + lax.axis_index("c")
        base = wid * _ROWS_PER_W

        def chunk(j, carry):
            off = base + j * _GCH
            pltpu.sync_copy(tok_slot_hbm.at[pl.ds(off, _GCH)], idx_v)
            pltpu.async_copy(tokens_hbm.at[idx_v], rows_v, sem).wait()
            pltpu.sync_copy(rows_v, x_hbm.at[pl.ds(off, _GCH)])
            return carry

        lax.fori_loop(0, _ROWS_PER_W // _GCH, chunk, 0)

    @functools.partial(
        pl.kernel,
        out_type=jax.ShapeDtypeStruct((T, D), jnp.float32),
        mesh=mesh,
        compiler_params=pltpu.CompilerParams(needs_layout_passes=False),
        scratch_types=[pltpu.VMEM((2 * _CCH,), jnp.int32),
                       pltpu.VMEM((2 * _CCH, D), jnp.float32),
                       pltpu.VMEM((_CCH, D), jnp.float32),
                       pltpu.SemaphoreType.DMA],
    )
    def sc_combine(slot_hbm, h_hbm, out_hbm, idx_v, rows_v, out_v, sem):
        wid = lax.axis_index("s") * NC + lax.axis_index("c")
        base = wid * _TOK_PER_W

        def chunk(j, carry):
            toff = base + j * _CCH
            pltpu.sync_copy(slot_hbm.at[pl.ds(K * toff, K * _CCH)], idx_v)
            pltpu.async_copy(h_hbm.at[idx_v], rows_v, sem).wait()

            def row(r, carry2):
                for c in range(D // LANES):
                    sl = pl.ds(c * LANES, LANES)
                    out_v[r, sl] = rows_v[2 * r, sl] + rows_v[2 * r + 1, sl]
                return carry2

            lax.fori_loop(0, _CCH, row, 0)
            pltpu.sync_copy(out_v, out_hbm.at[pl.ds(toff, _CCH)])
            return carry

        lax.fori_loop(0, _TOK_PER_W // _CCH, chunk, 0)

    return sc_invert, sc_gather_tokens, sc_combine


@jax.jit
def kernel(hidden_states, Wg, bg, Wn, bn, W_in, b_in, W_out, b_out):
    B, S, _ = hidden_states.shape
    tokens = hidden_states.reshape(-1, D)
    noise = jax.random.normal(jax.random.key(42), (T, E), dtype=jnp.float32)

    gate, noisy, slot_tk, w_tk, eb, nbtot = _run_router(
        tokens, Wg, bg, Wn, bn, noise)

    sc_invert, sc_gather_tokens, sc_combine = _sc_kernels()
    slot_flat = slot_tk.reshape(NP)
    tok_slot, w_slot = sc_invert(slot_flat, w_tk.reshape(NP))
    x_sorted = sc_gather_tokens(tok_slot, tokens)

    h = _run_gmm(eb.reshape(NB_MAX), nbtot.reshape(1), x_sorted,
                 w_slot.reshape(NSLOTS, 1), W_in.astype(jnp.bfloat16),
                 b_in.reshape(E, 1, FF), W_out.astype(jnp.bfloat16),
                 b_out.reshape(E, 1, D))

    out = sc_combine(slot_flat, h)
    return (out.reshape(B, S, D), noisy, gate)


# speedup vs baseline: 2.5691x; 1.2274x over previous
import functools

import jax
import jax.numpy as jnp
from jax import lax
from jax.experimental import pallas as pl
from jax.experimental.pallas import tpu as pltpu
from jax.experimental.pallas import tpu_sc as plsc

D = 1024
FF = 4096
E = 8
K = 2
T = 2048
TM = 256
NB_MAX = (T * K) // TM + E
NSLOTS = NB_MAX * TM
FJ = 512
NP = T * K
_INV_SQRT2 = 0.7071067811865476

NC, NS, LANES = 2, 16, 16
NW = NC * NS
_GCH = 64
_ROWS_PER_W = NSLOTS // NW
_CCH = 32
_TOK_PER_W = T // NW


def _router_body(tokens_ref, wg_ref, bg_ref, wn_ref, bn_ref, noise_ref,
                 gate_ref, noisy_ref, slot_ref, w_ref, eb_ref, nbtot_ref):
    x = tokens_ref[...]
    gate = jnp.dot(x, wg_ref[...], preferred_element_type=jnp.float32) + bg_ref[...]
    nl = jnp.dot(x, wn_ref[...], preferred_element_type=jnp.float32) + bn_ref[...]
    sp = jnp.maximum(nl, 0.0) + jnp.log1p(jnp.exp(-jnp.abs(nl)))
    noisy = gate + noise_ref[...] * sp
    gate_ref[...] = gate
    noisy_ref[...] = noisy
    m = jnp.max(noisy, axis=1, keepdims=True)
    ex = jnp.exp(noisy - m)
    rw = ex / jnp.sum(ex, axis=1, keepdims=True)
    lane = lax.broadcasted_iota(jnp.int32, rw.shape, 1)
    m1 = jnp.max(rw, axis=1, keepdims=True)
    idx1 = jnp.min(jnp.where(rw == m1, lane, E), axis=1, keepdims=True)
    sel1 = lane == idx1
    rw_rest = jnp.where(sel1, -jnp.inf, rw)
    m2 = jnp.max(rw_rest, axis=1, keepdims=True)
    idx2 = jnp.min(jnp.where(rw_rest == m2, lane, E), axis=1, keepdims=True)
    sel2 = lane == idx2
    denom = m1 + m2
    w_ref[...] = jnp.concatenate([m1 / denom, m2 / denom], axis=1)

    sel = (sel1 | sel2).astype(jnp.bfloat16)
    r_i = lax.broadcasted_iota(jnp.int32, (T, T), 0)
    c_i = lax.broadcasted_iota(jnp.int32, (T, T), 1)
    ltri = (r_i >= c_i).astype(jnp.bfloat16)
    ranks = jnp.dot(ltri, sel, preferred_element_type=jnp.float32)
    counts = ranks[T - 1:T, :]
    nb = jnp.floor((counts + (TM - 1)) * (1.0 / TM))
    a_i = lax.broadcasted_iota(jnp.int32, (E, E), 0)
    b_i = lax.broadcasted_iota(jnp.int32, (E, E), 1)
    utri = (a_i <= b_i).astype(jnp.float32)
    nb_cum = jnp.dot(nb, utri, preferred_element_type=jnp.float32)
    base = (nb_cum - nb) * TM
    nbtot = nb_cum[0:1, E - 1:E]
    nbtot_ref[...] = nbtot.astype(jnp.int32)

    def pick(sel_k):
        rank_k = jnp.sum(jnp.where(sel_k, ranks, 0.0), axis=1, keepdims=True)
        base_k = jnp.sum(jnp.where(sel_k, base, 0.0), axis=1, keepdims=True)
        return (base_k + rank_k - 1.0).astype(jnp.int32)

    slot_ref[...] = jnp.concatenate([pick(sel1), pick(sel2)], axis=1)

    ends = nb_cum * TM
    bb = lax.broadcasted_iota(jnp.int32, (NB_MAX, 1), 0).astype(jnp.float32)
    bmin = jnp.minimum(bb, nbtot - 1.0) * TM
    eb_ref[...] = jnp.sum((ends <= bmin).astype(jnp.int32), axis=1,
                          keepdims=True)


def _run_router(tokens, Wg, bg, Wn, bn, noise):
    return pl.pallas_call(
        _router_body,
        out_shape=(
            jax.ShapeDtypeStruct((T, E), jnp.float32),
            jax.ShapeDtypeStruct((T, E), jnp.float32),
            jax.ShapeDtypeStruct((T, K), jnp.int32),
            jax.ShapeDtypeStruct((T, K), jnp.float32),
            jax.ShapeDtypeStruct((NB_MAX, 1), jnp.int32),
            jax.ShapeDtypeStruct((1, 1), jnp.int32),
        ),
    )(tokens, Wg, bg.reshape(1, E), Wn, bn.reshape(1, E), noise)


def _gmm_body(eb_ref, nbtot_ref, toks_ref, tsl_ref, wps_ref, win_ref, bin_ref,
              wout_ref, bout_ref, h_ref, acc_ref):
    b = pl.program_id(0)

    @pl.when(b < nbtot_ref[0])
    def _():
        cols = lax.broadcasted_iota(jnp.int32, (TM, T), 1)
        p = (cols == tsl_ref[...]).astype(jnp.bfloat16)
        x = jnp.dot(p, toks_ref[...],
                    preferred_element_type=jnp.float32).astype(jnp.bfloat16)

        def body(j, carry):
            sl = pl.ds(j * FJ, FJ)
            h1 = jnp.dot(x, win_ref[0, :, sl],
                         preferred_element_type=jnp.float32)
            h1 = h1 + bin_ref[0, 0, sl][None, :]
            h1 = h1 * 0.5 * (1.0 + lax.erf(h1 * _INV_SQRT2))
            acc = jnp.dot(h1.astype(jnp.bfloat16), wout_ref[0, sl, :],
                          preferred_element_type=jnp.float32)

            @pl.when(j == 0)
            def _():
                acc_ref[...] = acc

            @pl.when(j > 0)
            def _():
                acc_ref[...] += acc

            return carry

        lax.fori_loop(0, FF // FJ, body, 0)
        h_ref[...] = wps_ref[...] * (acc_ref[...] + bout_ref[0, 0, :][None, :])


def _run_gmm(eb, nbtot, tokens_bf, tok_slot, w_slot, win_bf, b_in3, wout_bf,
             b_out3):
    grid_spec = pltpu.PrefetchScalarGridSpec(
        num_scalar_prefetch=2,
        grid=(NB_MAX,),
        in_specs=[
            pl.BlockSpec((T, D), lambda b, eb, nt: (0, 0)),
            pl.BlockSpec((TM, 1), lambda b, eb, nt: (b, 0)),
            pl.BlockSpec((TM, 1), lambda b, eb, nt: (b, 0)),
            pl.BlockSpec((1, D, FF), lambda b, eb, nt: (eb[b], 0, 0)),
            pl.BlockSpec((1, 1, FF), lambda b, eb, nt: (eb[b], 0, 0)),
            pl.BlockSpec((1, FF, D), lambda b, eb, nt: (eb[b], 0, 0)),
            pl.BlockSpec((1, 1, D), lambda b, eb, nt: (eb[b], 0, 0)),
        ],
        out_specs=pl.BlockSpec((TM, D), lambda b, eb, nt: (b, 0)),
        scratch_shapes=[pltpu.VMEM((TM, D), jnp.float32)],
    )
    return pl.pallas_call(
        _gmm_body,
        grid_spec=grid_spec,
        out_shape=jax.ShapeDtypeStruct((NSLOTS, D), jnp.float32),
    )(eb, nbtot, tokens_bf, tok_slot, w_slot, win_bf, b_in3, wout_bf, b_out3)


@functools.cache
def _sc_kernels():
    mesh = plsc.VectorSubcoreMesh(core_axis_name="c", subcore_axis_name="s",
                                  num_cores=NC, num_subcores=NS)

    @functools.partial(
        pl.kernel,
        out_type=(jax.ShapeDtypeStruct((NSLOTS,), jnp.int32),
                  jax.ShapeDtypeStruct((NSLOTS,), jnp.float32)),
        mesh=mesh,
        compiler_params=pltpu.CompilerParams(needs_layout_passes=False),
        scratch_types=[pltpu.VMEM((NP,), jnp.int32),
                       pltpu.VMEM((NP,), jnp.float32),
                       pltpu.VMEM((NSLOTS,), jnp.int32),
                       pltpu.VMEM((NSLOTS,), jnp.float32)],
    )
    def sc_invert(slot_hbm, w_hbm, tok_hbm, wslot_hbm, slot_v, w_v, tok_v,
                  wslot_v):
        cid = lax.axis_index("c")
        sid = lax.axis_index("s")

        @pl.when(jnp.logical_and(cid == 0, sid == 0))
        def _():
            pltpu.sync_copy(slot_hbm, slot_v)
            pltpu.sync_copy(w_hbm, w_v)
            zi = jnp.zeros((LANES,), jnp.int32)
            zf = jnp.zeros((LANES,), jnp.float32)

            def init(i, carry):
                tok_v[pl.ds(i * LANES, LANES)] = zi
                wslot_v[pl.ds(i * LANES, LANES)] = zf
                return carry

            lax.fori_loop(0, NSLOTS // LANES, init, 0)
            lanes = lax.iota(jnp.int32, LANES)

            def body(i, carry):
                sl = pl.ds(i * LANES, LANES)
                idx = slot_v[sl]
                plsc.store_scatter(tok_v, [idx], (i * LANES + lanes) >> 1)
                plsc.store_scatter(wslot_v, [idx], w_v[sl])
                return carry

            lax.fori_loop(0, NP // LANES, body, 0)
            pltpu.sync_copy(tok_v, tok_hbm)
            pltpu.sync_copy(wslot_v, wslot_hbm)

    @functools.partial(
        pl.kernel,
        out_type=jax.ShapeDtypeStruct((T, D), jnp.float32),
        mesh=mesh,
        compiler_params=pltpu.CompilerParams(needs_layout_passes=False),
        scratch_types=[pltpu.VMEM((2 * _CCH,), jnp.int32),
                       pltpu.VMEM((2 * _CCH, D), jnp.float32),
                       pltpu.VMEM((_CCH, D), jnp.float32),
                       pltpu.SemaphoreType.DMA],
    )
    def sc_combine(slot_hbm, h_hbm, out_hbm, idx_v, rows_v, out_v, sem):
        wid = lax.axis_index("s") * NC + lax.axis_index("c")
        base = wid * _TOK_PER_W

        def chunk(j, carry):
            toff = base + j * _CCH
            pltpu.sync_copy(slot_hbm.at[pl.ds(K * toff, K * _CCH)], idx_v)
            pltpu.async_copy(h_hbm.at[idx_v], rows_v, sem).wait()

            def row(r, carry2):
                for c in range(D // LANES):
                    sl = pl.ds(c * LANES, LANES)
                    out_v[r, sl] = rows_v[2 * r, sl] + rows_v[2 * r + 1, sl]
                return carry2

            lax.fori_loop(0, _CCH, row, 0)
            pltpu.sync_copy(out_v, out_hbm.at[pl.ds(toff, _CCH)])
            return carry

        lax.fori_loop(0, _TOK_PER_W // _CCH, chunk, 0)

    return sc_invert, sc_combine


@jax.jit
def kernel(hidden_states, Wg, bg, Wn, bn, W_in, b_in, W_out, b_out):
    B, S, _ = hidden_states.shape
    tokens = hidden_states.reshape(-1, D)
    noise = jax.random.normal(jax.random.key(42), (T, E), dtype=jnp.float32)

    gate, noisy, slot_tk, w_tk, eb, nbtot = _run_router(
        tokens, Wg, bg, Wn, bn, noise)

    sc_invert, sc_combine = _sc_kernels()
    slot_flat = slot_tk.reshape(NP)
    tok_slot, w_slot = sc_invert(slot_flat, w_tk.reshape(NP))

    h = _run_gmm(eb.reshape(NB_MAX), nbtot.reshape(1),
                 tokens.astype(jnp.bfloat16), tok_slot.reshape(NSLOTS, 1),
                 w_slot.reshape(NSLOTS, 1), W_in.astype(jnp.bfloat16),
                 b_in.reshape(E, 1, FF), W_out.astype(jnp.bfloat16),
                 b_out.reshape(E, 1, D))

    out = sc_combine(slot_flat, h)
    return (out.reshape(B, S, D), noisy, gate)
